# baseline (speedup 1.0000x reference)
import functools

import jax
import jax.numpy as jnp
from jax import lax
from jax.experimental import pallas as pl
from jax.experimental.pallas import tpu as pltpu
from jax.experimental.pallas import tpu_sc as plsc

_N = 320000
_D = 128
_M = 10000
_H = 128
_NC = 2
_NS = 16
_NW = _NC * _NS

_NSC = 179200
_RPW = _NSC // _NW
_CH = 128
_NCH = _RPW // _CH
_TAIL = _RPW - _NCH * _CH
_MROWS = 632
_MP = _MROWS * _NS


def _sc_body(x_hbm, idx_hbm, out0_hbm, out1_hbm, acc, rows0, rows1, ia0, ia1,
             itail, idxall, zbuf, rsem0, rsem1, isem0, isem1):
  c = lax.axis_index("c")
  s = lax.axis_index("s")
  wid = s * _NC + c
  base = wid * _RPW

  rows = (rows0, rows1)
  ia = (ia0, ia1)
  rsem = (rsem0, rsem1)

  def fire(j, b):
    off = base + j * _CH
    pltpu.async_copy(x_hbm.at[pl.ds(off, _CH)], rows[b], rsem[b])

  def wait(b):
    pltpu.make_async_copy(x_hbm.at[pl.ds(0, _CH)], rows[b], rsem[b]).wait()

  def stage_ids(j, b):
    def cp(k, carry):
      ia[b][pl.ds(k * 16, 16)] = idxall[pl.ds(j * _CH + k * 16, 16)]
      return carry
    lax.fori_loop(0, _CH // 16, cp, 0)

  def scatter(b):
    pltpu.sync_copy(rows[b], acc.at[ia[b]], add=True)

  fire(0, 0)
  fire(1, 1)
  pltpu.async_copy(idx_hbm.at[pl.ds(base, _RPW)], idxall.at[pl.ds(0, _RPW)],
                   isem0)

  def zloop(k, carry):
    r = k // 8
    col = (k % 8) * 16
    zbuf[r, pl.ds(col, 16)] = jnp.zeros((16,), jnp.float32)
    return carry
  lax.fori_loop(0, 32 * 8, zloop, 0)
  r0 = s * _MROWS
  for t in range(0, (_MROWS // 32) * 32, 32):
    pltpu.sync_copy(zbuf, acc.at[pl.ds(r0 + t, 32)])
  rem = _MROWS % 32
  if rem:
    pltpu.sync_copy(zbuf.at[pl.ds(0, rem)],
                    acc.at[pl.ds(r0 + (_MROWS // 32) * 32, rem)])
  plsc.subcore_barrier()

  pltpu.make_async_copy(idx_hbm.at[pl.ds(0, _RPW)], idxall.at[pl.ds(0, _RPW)],
                        isem0).wait()

  def lbody(jj, carry):
    for b in range(2):
      j = jj * 2 + b
      stage_ids(j, b)
      wait(b)
      scatter(b)
      if b == 0:
        fire(j + 2, b)
      else:
        @pl.when(jj < _NCH // 2 - 1)
        def _():
          fire(j + 2, b)
    return carry
  lax.fori_loop(0, _NCH // 2, lbody, 0)
  if _NCH % 2:
    stage_ids(_NCH - 1, 0)
    wait(0)
    scatter(0)
  if _TAIL:
    toff = base + _NCH * _CH
    pltpu.sync_copy(x_hbm.at[pl.ds(toff, _TAIL)], rows0.at[pl.ds(0, _TAIL)])
    def cpt(k, carry):
      itail[pl.ds(k * 16, 16)] = idxall[pl.ds(_NCH * _CH + k * 16, 16)]
      return carry
    lax.fori_loop(0, _TAIL // 16, cpt, 0)
    pltpu.sync_copy(rows0.at[pl.ds(0, _TAIL)], acc.at[itail], add=True)

  plsc.subcore_barrier()
  @pl.when(c == 0)
  def _():
    pltpu.sync_copy(acc.at[pl.ds(r0, _MROWS)], out0_hbm.at[pl.ds(r0, _MROWS)])
  @pl.when(c == 1)
  def _():
    pltpu.sync_copy(acc.at[pl.ds(r0, _MROWS)], out1_hbm.at[pl.ds(r0, _MROWS)])


_sc_segment_sum = functools.partial(
    pl.kernel,
    out_type=(jax.ShapeDtypeStruct((_MP, _D), jnp.float32),
              jax.ShapeDtypeStruct((_MP, _D), jnp.float32)),
    mesh=plsc.VectorSubcoreMesh(core_axis_name="c", subcore_axis_name="s"),
    scratch_types=[
        pltpu.VMEM_SHARED((_MP, _D), jnp.float32),
        pltpu.VMEM((_CH, _D), jnp.float32),
        pltpu.VMEM((_CH, _D), jnp.float32),
        pltpu.VMEM((_CH,), jnp.int32),
        pltpu.VMEM((_CH,), jnp.int32),
        pltpu.VMEM((_TAIL,), jnp.int32),
        pltpu.VMEM((_RPW + 16,), jnp.int32),
        pltpu.VMEM((32, _D), jnp.float32),
        pltpu.SemaphoreType.DMA,
        pltpu.SemaphoreType.DMA,
        pltpu.SemaphoreType.DMA,
        pltpu.SemaphoreType.DMA,
    ],
)(_sc_body)


_BT = 512
_NTC = _N - _NSC
_NBT = _NTC // _BT
_W = 128
_MP2 = 10120
_IDG = _BT // 128


def _tc_scatter_body(x_ref, ids_ref, o_ref):
  i = pl.program_id(0)

  @pl.when(i == 0)
  def _():
    o_ref[...] = jnp.zeros((_MP2, _D), jnp.float32)

  ids = ids_ref[...].reshape(_IDG, 128)
  w0 = jnp.bitwise_and(jnp.min(ids), -8)
  loc = ids - w0

  def onehot_mm(loc2, valid):
    part = jnp.zeros((_W, _D), jnp.float32)
    for q in range(_IDG):
      oh = (lax.broadcasted_iota(jnp.int32, (_W, 128), 0) == loc2[q][None, :])
      oh = jnp.logical_and(oh, valid[q][None, :])
      part = part + jnp.dot(oh.astype(jnp.float32),
                            x_ref[pl.ds(q * 128, 128), :],
                            preferred_element_type=jnp.float32)
    return part

  part = onehot_mm(loc, jnp.full((_IDG, 128), True))
  o_ref[pl.ds(w0, _W), :] = o_ref[pl.ds(w0, _W), :] + part

  rem = (loc >= _W).astype(jnp.int32)

  def cond(carry):
    return jnp.max(carry) > 0

  def body(carry):
    sel = carry > 0
    w1 = jnp.bitwise_and(jnp.min(jnp.where(sel, ids, jnp.int32(2 ** 30))),
                         -8)
    loc2 = ids - w1
    part2 = onehot_mm(loc2, sel)
    o_ref[pl.ds(w1, _W), :] = o_ref[pl.ds(w1, _W), :] + part2
    return carry * (loc2 >= _W).astype(jnp.int32)

  lax.while_loop(cond, body, rem)


_tc_scatter = pl.pallas_call(
    _tc_scatter_body,
    grid=(_NBT,),
    in_specs=[
        pl.BlockSpec((_BT, _D), lambda i: (i + _NSC // _BT, 0)),
        pl.BlockSpec((_IDG, 1, 128), lambda i: (i + _NSC // _BT, 0, 0)),
    ],
    out_specs=pl.BlockSpec((_MP2, _D), lambda i: (0, 0)),
    out_shape=jax.ShapeDtypeStruct((_MP2, _D), jnp.float32),
)


_BM = 2000


def _tc_body(p0_ref, p1_ref, pt_ref, w1_ref, b1_ref, w2_ref, b2_ref, o_ref):
  agg = p0_ref[...] + p1_ref[...] + pt_ref[...]
  h = jnp.dot(agg, w1_ref[...].T, preferred_element_type=jnp.float32)
  h = h + b1_ref[...]
  h = h * jax.nn.sigmoid(h)
  y = jnp.sum(h * w2_ref[...], axis=1, keepdims=True) + b2_ref[...]
  o_ref[...] = y


_tc_mlp = pl.pallas_call(
    _tc_body,
    grid=(_M // _BM,),
    in_specs=[
        pl.BlockSpec((_BM, _D), lambda i: (i, 0)),
        pl.BlockSpec((_BM, _D), lambda i: (i, 0)),
        pl.BlockSpec((_BM, _D), lambda i: (i, 0)),
        pl.BlockSpec((_H, _D), lambda i: (0, 0)),
        pl.BlockSpec((1, _H), lambda i: (0, 0)),
        pl.BlockSpec((1, _H), lambda i: (0, 0)),
        pl.BlockSpec((1, 1), lambda i: (0, 0)),
    ],
    out_specs=pl.BlockSpec((_BM, 1), lambda i: (i, 0)),
    out_shape=jax.ShapeDtypeStruct((_M, 1), jnp.float32),
)


def kernel(scalar_representation, idx_m, W1, b1, W2, b2):
  p0, p1 = _sc_segment_sum(scalar_representation, idx_m)
  pt = _tc_scatter(scalar_representation, idx_m.reshape(_N // 128, 1, 128))
  return _tc_mlp(p0, p1, pt, W1, b1.reshape(1, _H), W2, b2.reshape(1, 1))


# speedup vs baseline: 1.5458x; 1.5458x over previous
import functools

import jax
import jax.numpy as jnp
from jax import lax
from jax.experimental import pallas as pl
from jax.experimental.pallas import tpu as pltpu
from jax.experimental.pallas import tpu_sc as plsc

_N = 320000
_D = 128
_M = 10000
_H = 128
_NC = 2
_NS = 16
_NW = _NC * _NS

_NSC = 140800
_RPW = _NSC // _NW
_CH = 128
_NCH = _RPW // _CH
_TAIL = _RPW - _NCH * _CH
_MROWS = 632
_MP = _MROWS * _NS


def _sc_body(x_hbm, idx_hbm, out0_hbm, out1_hbm, acc, rows0, rows1, ia0, ia1,
             itail, idxall, zbuf, rsem0, rsem1, isem0, isem1):
  c = lax.axis_index("c")
  s = lax.axis_index("s")
  wid = s * _NC + c
  base = wid * _RPW

  rows = (rows0, rows1)
  ia = (ia0, ia1)
  rsem = (rsem0, rsem1)

  def fire(j, b):
    off = base + j * _CH
    pltpu.async_copy(x_hbm.at[pl.ds(off, _CH)], rows[b], rsem[b])

  def wait(b):
    pltpu.make_async_copy(x_hbm.at[pl.ds(0, _CH)], rows[b], rsem[b]).wait()

  def stage_ids(j, b):
    def cp(k, carry):
      ia[b][pl.ds(k * 16, 16)] = idxall[pl.ds(j * _CH + k * 16, 16)]
      return carry
    lax.fori_loop(0, _CH // 16, cp, 0)

  def scatter(b):
    pltpu.sync_copy(rows[b], acc.at[ia[b]], add=True)

  fire(0, 0)
  fire(1, 1)
  pltpu.async_copy(idx_hbm.at[pl.ds(base, _RPW)], idxall.at[pl.ds(0, _RPW)],
                   isem0)

  def zloop(k, carry):
    r = k // 8
    col = (k % 8) * 16
    zbuf[r, pl.ds(col, 16)] = jnp.zeros((16,), jnp.float32)
    return carry
  lax.fori_loop(0, 32 * 8, zloop, 0)
  r0 = s * _MROWS
  for t in range(0, (_MROWS // 32) * 32, 32):
    pltpu.sync_copy(zbuf, acc.at[pl.ds(r0 + t, 32)])
  rem = _MROWS % 32
  if rem:
    pltpu.sync_copy(zbuf.at[pl.ds(0, rem)],
                    acc.at[pl.ds(r0 + (_MROWS // 32) * 32, rem)])
  plsc.subcore_barrier()

  pltpu.make_async_copy(idx_hbm.at[pl.ds(0, _RPW)], idxall.at[pl.ds(0, _RPW)],
                        isem0).wait()

  def lbody(jj, carry):
    for b in range(2):
      j = jj * 2 + b
      stage_ids(j, b)
      wait(b)
      scatter(b)
      @pl.when(j + 2 < _NCH)
      def _():
        fire(j + 2, b)
    return carry
  lax.fori_loop(0, _NCH // 2, lbody, 0)
  if _NCH % 2:
    stage_ids(_NCH - 1, 0)
    wait(0)
    scatter(0)
  if _TAIL:
    toff = base + _NCH * _CH
    pltpu.sync_copy(x_hbm.at[pl.ds(toff, _TAIL)], rows0.at[pl.ds(0, _TAIL)])
    def cpt(k, carry):
      itail[pl.ds(k * 16, 16)] = idxall[pl.ds(_NCH * _CH + k * 16, 16)]
      return carry
    lax.fori_loop(0, _TAIL // 16, cpt, 0)
    pltpu.sync_copy(rows0.at[pl.ds(0, _TAIL)], acc.at[itail], add=True)

  plsc.subcore_barrier()
  @pl.when(c == 0)
  def _():
    pltpu.sync_copy(acc.at[pl.ds(r0, _MROWS)], out0_hbm.at[pl.ds(r0, _MROWS)])
  @pl.when(c == 1)
  def _():
    pltpu.sync_copy(acc.at[pl.ds(r0, _MROWS)], out1_hbm.at[pl.ds(r0, _MROWS)])


_sc_segment_sum = functools.partial(
    pl.kernel,
    out_type=(jax.ShapeDtypeStruct((_MP, _D), jnp.float32),
              jax.ShapeDtypeStruct((_MP, _D), jnp.float32)),
    mesh=plsc.VectorSubcoreMesh(core_axis_name="c", subcore_axis_name="s"),
    scratch_types=[
        pltpu.VMEM_SHARED((_MP, _D), jnp.float32),
        pltpu.VMEM((_CH, _D), jnp.float32),
        pltpu.VMEM((_CH, _D), jnp.float32),
        pltpu.VMEM((_CH,), jnp.int32),
        pltpu.VMEM((_CH,), jnp.int32),
        pltpu.VMEM((_TAIL,), jnp.int32),
        pltpu.VMEM((_RPW + 16,), jnp.int32),
        pltpu.VMEM((32, _D), jnp.float32),
        pltpu.SemaphoreType.DMA,
        pltpu.SemaphoreType.DMA,
        pltpu.SemaphoreType.DMA,
        pltpu.SemaphoreType.DMA,
    ],
)(_sc_body)


_BT = 1280
_NTC = _N - _NSC
_NBT = _NTC // _BT
_W = 64
_MP2 = 10056
_IDG = _BT // 128


def _tc_scatter_body(x_ref, ids_ref, o_ref):
  i = pl.program_id(0)

  @pl.when(i == 0)
  def _():
    o_ref[...] = jnp.zeros((_MP2, _D), jnp.float32)

  ids = ids_ref[...].reshape(_IDG, 128)
  w0 = jnp.bitwise_and(jnp.min(ids), -8)
  loc = ids - w0

  def onehot_mm(loc2, valid):
    part = jnp.zeros((_W, _D), jnp.float32)
    for q in range(_IDG):
      oh = (lax.broadcasted_iota(jnp.int32, (_W, 128), 0) == loc2[q][None, :])
      oh = jnp.logical_and(oh, valid[q][None, :])
      part = part + jnp.dot(oh.astype(jnp.float32),
                            x_ref[pl.ds(q * 128, 128), :],
                            preferred_element_type=jnp.float32)
    return part

  part = onehot_mm(loc, jnp.full((_IDG, 128), True))
  o_ref[pl.ds(w0, _W), :] = o_ref[pl.ds(w0, _W), :] + part

  rem = (loc >= _W).astype(jnp.int32)

  def cond(carry):
    return jnp.max(carry) > 0

  def body(carry):
    sel = carry > 0
    w1 = jnp.bitwise_and(jnp.min(jnp.where(sel, ids, jnp.int32(2 ** 30))),
                         -8)
    loc2 = ids - w1
    part2 = onehot_mm(loc2, sel)
    o_ref[pl.ds(w1, _W), :] = o_ref[pl.ds(w1, _W), :] + part2
    return carry * (loc2 >= _W).astype(jnp.int32)

  lax.while_loop(cond, body, rem)


_tc_scatter = pl.pallas_call(
    _tc_scatter_body,
    grid=(_NBT,),
    in_specs=[
        pl.BlockSpec((_BT, _D), lambda i: (i + _NSC // _BT, 0)),
        pl.BlockSpec((_IDG, 1, 128), lambda i: (i + _NSC // _BT, 0, 0)),
    ],
    out_specs=pl.BlockSpec((_MP2, _D), lambda i: (0, 0)),
    out_shape=jax.ShapeDtypeStruct((_MP2, _D), jnp.float32),
)


_BM = 2000


def _tc_body(p0_ref, p1_ref, pt_ref, w1_ref, b1_ref, w2_ref, b2_ref, o_ref):
  agg = p0_ref[...] + p1_ref[...] + pt_ref[...]
  h = jnp.dot(agg, w1_ref[...].T, preferred_element_type=jnp.float32)
  h = h + b1_ref[...]
  h = h * jax.nn.sigmoid(h)
  y = jnp.sum(h * w2_ref[...], axis=1, keepdims=True) + b2_ref[...]
  o_ref[...] = y


_tc_mlp = pl.pallas_call(
    _tc_body,
    grid=(_M // _BM,),
    in_specs=[
        pl.BlockSpec((_BM, _D), lambda i: (i, 0)),
        pl.BlockSpec((_BM, _D), lambda i: (i, 0)),
        pl.BlockSpec((_BM, _D), lambda i: (i, 0)),
        pl.BlockSpec((_H, _D), lambda i: (0, 0)),
        pl.BlockSpec((1, _H), lambda i: (0, 0)),
        pl.BlockSpec((1, _H), lambda i: (0, 0)),
        pl.BlockSpec((1, 1), lambda i: (0, 0)),
    ],
    out_specs=pl.BlockSpec((_BM, 1), lambda i: (i, 0)),
    out_shape=jax.ShapeDtypeStruct((_M, 1), jnp.float32),
)


def kernel(scalar_representation, idx_m, W1, b1, W2, b2):
  p0, p1 = _sc_segment_sum(scalar_representation, idx_m)
  pt = _tc_scatter(scalar_representation, idx_m.reshape(_N // 128, 1, 128))
  return _tc_mlp(p0, p1, pt, W1, b1.reshape(1, _H), W2, b2.reshape(1, 1))


# speedup vs baseline: 1.5626x; 1.0109x over previous
import functools

import jax
import jax.numpy as jnp
from jax import lax
from jax.experimental import pallas as pl
from jax.experimental.pallas import tpu as pltpu
from jax.experimental.pallas import tpu_sc as plsc

_N = 320000
_D = 128
_M = 10000
_H = 128
_NC = 2
_NS = 16
_NW = _NC * _NS

_NSC = 140800
_RPW = _NSC // _NW
_CH = 128
_NCH = _RPW // _CH
_TAIL = _RPW - _NCH * _CH
_MROWS = 632
_MP = _MROWS * _NS


def _sc_body(x_hbm, idx_hbm, out0_hbm, out1_hbm, acc, rows0, rows1, ia0, ia1,
             itail, idxall, zbuf, rsem0, rsem1, isem0, isem1):
  c = lax.axis_index("c")
  s = lax.axis_index("s")
  wid = s * _NC + c
  base = wid * _RPW

  rows = (rows0, rows1)
  ia = (ia0, ia1)
  rsem = (rsem0, rsem1)

  def fire(j, b):
    off = base + j * _CH
    pltpu.async_copy(x_hbm.at[pl.ds(off, _CH)], rows[b], rsem[b])

  def wait(b):
    pltpu.make_async_copy(x_hbm.at[pl.ds(0, _CH)], rows[b], rsem[b]).wait()

  def stage_ids(j, b):
    def cp(k, carry):
      ia[b][pl.ds(k * 16, 16)] = idxall[pl.ds(j * _CH + k * 16, 16)]
      return carry
    lax.fori_loop(0, _CH // 16, cp, 0)

  def scatter(b):
    pltpu.sync_copy(rows[b], acc.at[ia[b]], add=True)

  fire(0, 0)
  fire(1, 1)
  pltpu.async_copy(idx_hbm.at[pl.ds(base, _RPW)], idxall.at[pl.ds(0, _RPW)],
                   isem0)

  def zloop(k, carry):
    r = k // 8
    col = (k % 8) * 16
    zbuf[r, pl.ds(col, 16)] = jnp.zeros((16,), jnp.float32)
    return carry
  lax.fori_loop(0, 32 * 8, zloop, 0)
  r0 = s * _MROWS
  for t in range(0, (_MROWS // 32) * 32, 32):
    pltpu.sync_copy(zbuf, acc.at[pl.ds(r0 + t, 32)])
  rem = _MROWS % 32
  if rem:
    pltpu.sync_copy(zbuf.at[pl.ds(0, rem)],
                    acc.at[pl.ds(r0 + (_MROWS // 32) * 32, rem)])
  plsc.subcore_barrier()

  pltpu.make_async_copy(idx_hbm.at[pl.ds(0, _RPW)], idxall.at[pl.ds(0, _RPW)],
                        isem0).wait()

  def lbody(jj, carry):
    for b in range(2):
      j = jj * 2 + b
      stage_ids(j, b)
      wait(b)
      scatter(b)
      @pl.when(j + 2 < _NCH)
      def _():
        fire(j + 2, b)
    return carry
  lax.fori_loop(0, _NCH // 2, lbody, 0)
  if _NCH % 2:
    stage_ids(_NCH - 1, 0)
    wait(0)
    scatter(0)
  if _TAIL:
    toff = base + _NCH * _CH
    pltpu.sync_copy(x_hbm.at[pl.ds(toff, _TAIL)], rows0.at[pl.ds(0, _TAIL)])
    def cpt(k, carry):
      itail[pl.ds(k * 16, 16)] = idxall[pl.ds(_NCH * _CH + k * 16, 16)]
      return carry
    lax.fori_loop(0, _TAIL // 16, cpt, 0)
    pltpu.sync_copy(rows0.at[pl.ds(0, _TAIL)], acc.at[itail], add=True)

  plsc.subcore_barrier()
  @pl.when(c == 0)
  def _():
    pltpu.sync_copy(acc.at[pl.ds(r0, _MROWS)], out0_hbm.at[pl.ds(r0, _MROWS)])
  @pl.when(c == 1)
  def _():
    pltpu.sync_copy(acc.at[pl.ds(r0, _MROWS)], out1_hbm.at[pl.ds(r0, _MROWS)])


_sc_segment_sum = functools.partial(
    pl.kernel,
    out_type=(jax.ShapeDtypeStruct((_MP, _D), jnp.float32),
              jax.ShapeDtypeStruct((_MP, _D), jnp.float32)),
    mesh=plsc.VectorSubcoreMesh(core_axis_name="c", subcore_axis_name="s"),
    scratch_types=[
        pltpu.VMEM_SHARED((_MP, _D), jnp.float32),
        pltpu.VMEM((_CH, _D), jnp.float32),
        pltpu.VMEM((_CH, _D), jnp.float32),
        pltpu.VMEM((_CH,), jnp.int32),
        pltpu.VMEM((_CH,), jnp.int32),
        pltpu.VMEM((_TAIL,), jnp.int32),
        pltpu.VMEM((_RPW + 16,), jnp.int32),
        pltpu.VMEM((32, _D), jnp.float32),
        pltpu.SemaphoreType.DMA,
        pltpu.SemaphoreType.DMA,
        pltpu.SemaphoreType.DMA,
        pltpu.SemaphoreType.DMA,
    ],
)(_sc_body)


_BT = 1280
_NTC = _N - _NSC
_NBT = _NTC // _BT
_W = 64
_MP2 = 10056
_IDG = _BT // 128


def _tc_scatter_body(x_ref, ids_ref, o_ref):
  i = pl.program_id(0)

  @pl.when(i == 0)
  def _():
    o_ref[...] = jnp.zeros((_MP2, _D), jnp.float32)

  ids = ids_ref[...].reshape(1, _BT)
  w0 = jnp.bitwise_and(jnp.min(ids), -8)
  loc = ids - w0

  def onehot_mm(loc2, valid):
    oh = (lax.broadcasted_iota(jnp.int32, (_W, _BT), 0) == loc2)
    oh = jnp.logical_and(oh, valid)
    return jnp.dot(oh.astype(jnp.float32), x_ref[...],
                   preferred_element_type=jnp.float32)

  part = onehot_mm(loc, jnp.full((1, _BT), True))
  o_ref[pl.ds(w0, _W), :] = o_ref[pl.ds(w0, _W), :] + part

  rem = (loc >= _W).astype(jnp.int32)

  def cond(carry):
    return jnp.max(carry) > 0

  def body(carry):
    sel = carry > 0
    w1 = jnp.bitwise_and(jnp.min(jnp.where(sel, ids, jnp.int32(2 ** 30))),
                         -8)
    loc2 = ids - w1
    part2 = onehot_mm(loc2, sel)
    o_ref[pl.ds(w1, _W), :] = o_ref[pl.ds(w1, _W), :] + part2
    return carry * (loc2 >= _W).astype(jnp.int32)

  lax.while_loop(cond, body, rem)


_tc_scatter = pl.pallas_call(
    _tc_scatter_body,
    grid=(_NBT,),
    in_specs=[
        pl.BlockSpec((_BT, _D), lambda i: (i + _NSC // _BT, 0)),
        pl.BlockSpec((1, 1, _BT), lambda i: (i + _NSC // _BT, 0, 0)),
    ],
    out_specs=pl.BlockSpec((_MP2, _D), lambda i: (0, 0)),
    out_shape=jax.ShapeDtypeStruct((_MP2, _D), jnp.float32),
)


_BM = 2000


def _tc_body(p0_ref, p1_ref, pt_ref, w1_ref, b1_ref, w2_ref, b2_ref, o_ref):
  agg = p0_ref[...] + p1_ref[...] + pt_ref[...]
  h = jnp.dot(agg, w1_ref[...].T, preferred_element_type=jnp.float32)
  h = h + b1_ref[...]
  h = h * jax.nn.sigmoid(h)
  y = jnp.sum(h * w2_ref[...], axis=1, keepdims=True) + b2_ref[...]
  o_ref[...] = y


_tc_mlp = pl.pallas_call(
    _tc_body,
    grid=(_M // _BM,),
    in_specs=[
        pl.BlockSpec((_BM, _D), lambda i: (i, 0)),
        pl.BlockSpec((_BM, _D), lambda i: (i, 0)),
        pl.BlockSpec((_BM, _D), lambda i: (i, 0)),
        pl.BlockSpec((_H, _D), lambda i: (0, 0)),
        pl.BlockSpec((1, _H), lambda i: (0, 0)),
        pl.BlockSpec((1, _H), lambda i: (0, 0)),
        pl.BlockSpec((1, 1), lambda i: (0, 0)),
    ],
    out_specs=pl.BlockSpec((_BM, 1), lambda i: (i, 0)),
    out_shape=jax.ShapeDtypeStruct((_M, 1), jnp.float32),
)


def kernel(scalar_representation, idx_m, W1, b1, W2, b2):
  p0, p1 = _sc_segment_sum(scalar_representation, idx_m)
  pt = _tc_scatter(scalar_representation, idx_m.reshape(_N // _BT, 1, _BT))
  return _tc_mlp(p0, p1, pt, W1, b1.reshape(1, _H), W2, b2.reshape(1, 1))
